# baseline (speedup 1.0000x reference)
import math
import functools

import jax
import jax.numpy as jnp
from jax.experimental import pallas as pl
from jax.experimental.pallas import tpu as pltpu

X_DIM = 128
N_TRAIN = 100000
KEY_DIM = 64
HIDDEN = 128
TIME_DIM = 32
TOPK = 64
BATCH = 1024

N_PAD = 100352
NB = 2048
GRID_N = N_PAD // NB
NEG_INF = float("-inf")


def _emb_mm_kernel(x_ref, t_ref, w_ref, b_ref, o_ref):
    half = TIME_DIM // 2
    i = jax.lax.broadcasted_iota(jnp.int32, (1, half), 1).astype(jnp.float32)
    freqs = jnp.exp((-math.log(10000.0)) * i / half)
    args = t_ref[:] * freqs
    h = jnp.concatenate([x_ref[:], jnp.sin(args), jnp.cos(args)], axis=-1)
    o_ref[:] = jax.lax.dot_general(
        h, w_ref[:], (((1,), (1,)), ((), ())),
        preferred_element_type=jnp.float32) + b_ref[:]


def _mm_kernel(h_ref, w_ref, b_ref, o_ref):
    o_ref[:] = jax.lax.dot_general(
        h_ref[:], w_ref[:], (((1,), (1,)), ((), ())),
        preferred_element_type=jnp.float32) + b_ref[:]


def _logits_kernel(q_ref, keys_ref, out_ref):
    i = pl.program_id(0)
    scale = 1.0 / math.sqrt(KEY_DIM)
    lg = jax.lax.dot_general(q_ref[:], keys_ref[:], (((1,), (1,)), ((), ())),
                             preferred_element_type=jnp.float32) * scale

    @pl.when(i == GRID_N - 1)
    def _():
        col = jax.lax.broadcasted_iota(jnp.int32, (BATCH, NB), 1)
        out_ref[:] = jnp.where(col >= NB - (N_PAD - N_TRAIN), NEG_INF, lg)

    @pl.when(i < GRID_N - 1)
    def _():
        out_ref[:] = lg


def _mm(h, W, b, n_out):
    return pl.pallas_call(
        _mm_kernel,
        out_shape=jax.ShapeDtypeStruct((BATCH, n_out), jnp.float32),
    )(h, W, b.reshape(1, n_out))


def _compute_q(x, t, W0, b0, W1, b1, W2, b2, W3, b3):
    y0 = pl.pallas_call(
        _emb_mm_kernel,
        out_shape=jax.ShapeDtypeStruct((BATCH, HIDDEN), jnp.float32),
    )(x, t.reshape(BATCH, 1), W0, b0.reshape(1, HIDDEN))
    h = jax.nn.selu(y0)
    h = jax.nn.selu(_mm(h, W1, b1, HIDDEN))
    h = jax.nn.selu(_mm(h, W2, b2, HIDDEN))
    return _mm(h, W3, b3, KEY_DIM)


def _compute_logits(q, keys_pad):
    return pl.pallas_call(
        _logits_kernel,
        grid=(GRID_N,),
        in_specs=[
            pl.BlockSpec((BATCH, KEY_DIM), lambda i: (0, 0)),
            pl.BlockSpec((NB, KEY_DIM), lambda i: (i, 0)),
        ],
        out_specs=pl.BlockSpec((BATCH, NB), lambda i: (0, i)),
        out_shape=jax.ShapeDtypeStruct((BATCH, N_PAD), jnp.float32),
    )(q, keys_pad)


def kernel(x, t, W0, b0, W1, b1, W2, b2, W3, b3, keys):
    q = _compute_q(x, t, W0, b0, W1, b1, W2, b2, W3, b3)
    keys_pad = jnp.pad(keys, ((0, N_PAD - N_TRAIN), (0, 0)))
    logits = _compute_logits(q, keys_pad)
    top_logits, top_idx = jax.lax.top_k(logits, TOPK)
    weights = jax.nn.softmax(top_logits, axis=-1)
    return weights, top_idx


# speedup vs baseline: 8.7577x; 8.7577x over previous
import math
import functools

import jax
import jax.numpy as jnp
from jax import lax
from jax.experimental import pallas as pl
from jax.experimental.pallas import tpu as pltpu
from jax.experimental.pallas import tpu_sc as plsc

X_DIM = 128
N_TRAIN = 100000
KEY_DIM = 64
HIDDEN = 128
TIME_DIM = 32
TOPK = 64
BATCH = 1024

N_PAD = 100352
NB = 2048
GRID_N = N_PAD // NB
NPIECE = N_PAD // 128
NEG_INF = float("-inf")

NSUB = 32
ROWS_PER = BATCH // NSUB
PIECE_CHUNK = 112
NCHUNK = NPIECE // PIECE_CHUNK
CAP = 1024
COMPACT_AT = 768
OUT_CAP = 128


def _emb_mm_kernel(x_ref, t_ref, w_ref, b_ref, o_ref):
    half = TIME_DIM // 2
    i = lax.broadcasted_iota(jnp.int32, (1, half), 1).astype(jnp.float32)
    freqs = jnp.exp((-math.log(10000.0)) * i / half)
    args = t_ref[:] * freqs
    h = jnp.concatenate([x_ref[:], jnp.sin(args), jnp.cos(args)], axis=-1)
    o_ref[:] = lax.dot_general(
        h, w_ref[:], (((1,), (1,)), ((), ())),
        preferred_element_type=jnp.float32) + b_ref[:]


def _mm_kernel(h_ref, w_ref, b_ref, o_ref):
    o_ref[:] = lax.dot_general(
        h_ref[:], w_ref[:], (((1,), (1,)), ((), ())),
        preferred_element_type=jnp.float32) + b_ref[:]


def _mm(h, W, b, n_out):
    return pl.pallas_call(
        _mm_kernel,
        out_shape=jax.ShapeDtypeStruct((BATCH, n_out), jnp.float32),
    )(h, W, b.reshape(1, n_out))


def _compute_q(x, t, W0, b0, W1, b1, W2, b2, W3, b3):
    y0 = pl.pallas_call(
        _emb_mm_kernel,
        out_shape=jax.ShapeDtypeStruct((BATCH, HIDDEN), jnp.float32),
    )(x, t.reshape(BATCH, 1), W0, b0.reshape(1, HIDDEN))
    h = jax.nn.selu(y0)
    h = jax.nn.selu(_mm(h, W1, b1, HIDDEN))
    h = jax.nn.selu(_mm(h, W2, b2, HIDDEN))
    return _mm(h, W3, b3, KEY_DIM)


def _logits3_kernel(q_ref, keys_ref, out_ref):
    i = pl.program_id(0)
    scale = 1.0 / math.sqrt(KEY_DIM)
    q = q_ref[:]
    for kk in range(16):
        lg = lax.dot_general(
            q, keys_ref[kk * 128:(kk + 1) * 128, :], (((1,), (1,)), ((), ())),
            preferred_element_type=jnp.float32) * scale
        out_ref[kk] = lg

    @pl.when(i == GRID_N - 1)
    def _():
        lane = lax.broadcasted_iota(jnp.int32, (BATCH, 128), 1)
        out_ref[13] = jnp.where(lane >= 32, NEG_INF, out_ref[13])
        out_ref[14] = jnp.full((BATCH, 128), NEG_INF, jnp.float32)
        out_ref[15] = jnp.full((BATCH, 128), NEG_INF, jnp.float32)


def _compute_logits3(q, keys_pad):
    return pl.pallas_call(
        _logits3_kernel,
        grid=(GRID_N,),
        in_specs=[
            pl.BlockSpec((BATCH, KEY_DIM), lambda i: (0, 0)),
            pl.BlockSpec((NB, KEY_DIM), lambda i: (i, 0)),
        ],
        out_specs=pl.BlockSpec((16, BATCH, 128), lambda i: (i, 0, 0)),
        out_shape=jax.ShapeDtypeStruct((NPIECE, BATCH, 128), jnp.float32),
    )(q, keys_pad)


def _sc_prune(cv, ci, cv2, ci2, cnt, stop_at):
    ninf = jnp.full((16,), NEG_INF, jnp.float32)
    pinf = jnp.full((16,), float("inf"), jnp.float32)

    nv0 = (cnt + 15) // 16

    def _mm_body(p, carry):
        vmax, vmin = carry
        v = cv[pl.ds(p * 16, 16)]
        vmax = jnp.maximum(vmax, v)
        vmin = jnp.minimum(vmin, jnp.where(v == NEG_INF, pinf, v))
        return vmax, vmin

    vmax, vmin = lax.fori_loop(0, nv0, _mm_body, (ninf, pinf))
    hi0 = jnp.max(vmax)
    lo0 = jnp.min(vmin)

    def _cond(st):
        lo, hi, c, k = st
        return jnp.logical_and(c > stop_at, k < 40)

    def _body(st):
        lo, hi, c, k = st
        mid = 0.5 * (lo + hi)
        mid_v = jnp.full((16,), 0.0, jnp.float32) + mid
        nv = (c + 15) // 16

        def _cnt_body(p, acc):
            v = cv[pl.ds(p * 16, 16)]
            n = jnp.sum((v >= mid_v).astype(jnp.int32))
            return acc + n

        count = lax.fori_loop(0, nv, _cnt_body, jnp.int32(0))

        def _refilter(_):
            def _fill(qq, _c):
                cv2[pl.ds(qq * 16, 16)] = ninf
                return _c
            lax.fori_loop(0, nv + 1, _fill, 0)

            def _cp(p, j):
                v = cv[pl.ds(p * 16, 16)]
                ix = ci[pl.ds(p * 16, 16)]
                m = v >= mid_v
                plsc.store_compressed(cv2.at[pl.ds(j, 16)], v, mask=m)
                plsc.store_compressed(ci2.at[pl.ds(j, 16)], ix, mask=m)
                return j + jnp.sum(m.astype(jnp.int32))

            j = lax.fori_loop(0, nv, _cp, jnp.int32(0))
            nv2 = (j + 15) // 16

            def _back(p, _c):
                cv[pl.ds(p * 16, 16)] = cv2[pl.ds(p * 16, 16)]
                ci[pl.ds(p * 16, 16)] = ci2[pl.ds(p * 16, 16)]
                return _c
            lax.fori_loop(0, nv2 + 1, _back, 0)
            return j

        new_c = lax.cond(count >= TOPK, _refilter, lambda _: c, 0)
        new_lo = jnp.where(count >= TOPK, mid, lo)
        new_hi = jnp.where(count >= TOPK, hi, mid)
        return new_lo, new_hi, new_c, k + 1

    lo, hi, cnt2, _ = lax.while_loop(_cond, _body, (lo0, hi0, cnt, jnp.int32(0)))
    return cnt2, lo


def _sc_topk_body(lg_hbm, outv_hbm, outi_hbm,
                  buf0, buf1, cv, ci, cv2, ci2, sem0, sem1):
    wid = lax.axis_index("s") * 2 + lax.axis_index("c")
    ninf = jnp.full((16,), NEG_INF, jnp.float32)
    iota = lax.iota(jnp.int32, 16)

    def _row_body(r, _carry):
        row = wid * ROWS_PER + r

        def _init(qq, _c):
            cv[pl.ds(qq * 16, 16)] = ninf
            return _c
        lax.fori_loop(0, CAP // 16, _init, 0)

        bufs = (buf0, buf1)
        sems = (sem0, sem1)
        h = pltpu.async_copy(
            lg_hbm.at[pl.ds(0, PIECE_CHUNK), row, :], buf0, sem0)
        state = (jnp.float32(NEG_INF), jnp.int32(0))
        for c in range(NCHUNK):
            cur = bufs[c % 2]
            if c + 1 < NCHUNK:
                h_next = pltpu.async_copy(
                    lg_hbm.at[pl.ds((c + 1) * PIECE_CHUNK, PIECE_CHUNK), row, :],
                    bufs[(c + 1) % 2], sems[(c + 1) % 2])
            h.wait()

            def _piece_body(p, st, cur=cur, c=c):
                tau, cnt = st
                tau_v = jnp.full((16,), 0.0, jnp.float32) + tau
                vs = [cur[p, pl.ds(k * 16, 16)] for k in range(8)]
                m01 = jnp.maximum(vs[0], vs[1])
                m23 = jnp.maximum(vs[2], vs[3])
                m45 = jnp.maximum(vs[4], vs[5])
                m67 = jnp.maximum(vs[6], vs[7])
                mx = jnp.maximum(jnp.maximum(m01, m23), jnp.maximum(m45, m67))
                has = jnp.max(mx)

                def _append(cnt0):
                    base = c * (PIECE_CHUNK * 128) + p * 128
                    cnt_l = cnt0
                    for k in range(8):
                        m = vs[k] > tau_v
                        ixv = jnp.full((16,), 0, jnp.int32) + base + k * 16 + iota
                        plsc.store_compressed(cv.at[pl.ds(cnt_l, 16)], vs[k], mask=m)
                        plsc.store_compressed(ci.at[pl.ds(cnt_l, 16)], ixv, mask=m)
                        cnt_l = cnt_l + jnp.sum(m.astype(jnp.int32))
                    cv[pl.ds(cnt_l, 16)] = ninf
                    return cnt_l

                cnt = lax.cond(has > tau, _append, lambda c0: c0, cnt)

                def _compact(args):
                    c0, t0 = args
                    c1, t1 = _sc_prune(cv, ci, cv2, ci2, c0, 96)
                    return c1, jnp.maximum(t0, t1)

                tau, cnt = lax.cond(
                    cnt >= COMPACT_AT,
                    lambda a: _compact(a)[::-1],
                    lambda a: (a[1], a[0]),
                    (cnt, tau))
                return tau, cnt

            state = lax.fori_loop(0, PIECE_CHUNK, _piece_body, state)
            if c + 1 < NCHUNK:
                h = h_next

        tau, cnt = state
        fcnt, _ftau = _sc_prune(cv, ci, cv2, ci2, cnt, 96)
        for off in range(4):
            cv[pl.ds(fcnt + off * 16, 16)] = ninf
        pltpu.sync_copy(cv.at[pl.ds(0, OUT_CAP)], outv_hbm.at[row])
        pltpu.sync_copy(ci.at[pl.ds(0, OUT_CAP)], outi_hbm.at[row])
        return _carry

    lax.fori_loop(0, ROWS_PER, _row_body, 0)


def _sc_topk(logits3):
    mesh = plsc.VectorSubcoreMesh(core_axis_name="c", subcore_axis_name="s")
    fn = pl.kernel(
        _sc_topk_body,
        out_type=(jax.ShapeDtypeStruct((BATCH, OUT_CAP), jnp.float32),
                  jax.ShapeDtypeStruct((BATCH, OUT_CAP), jnp.int32)),
        mesh=mesh,
        compiler_params=pltpu.CompilerParams(needs_layout_passes=False),
        scratch_types=[
            pltpu.VMEM((PIECE_CHUNK, 128), jnp.float32),
            pltpu.VMEM((PIECE_CHUNK, 128), jnp.float32),
            pltpu.VMEM((CAP,), jnp.float32),
            pltpu.VMEM((CAP,), jnp.int32),
            pltpu.VMEM((CAP,), jnp.float32),
            pltpu.VMEM((CAP,), jnp.int32),
            pltpu.SemaphoreType.DMA,
            pltpu.SemaphoreType.DMA,
        ],
    )
    return fn(logits3)


RB = 64


def _rank_kernel(cv_ref, ci_ref, w_ref, idx_ref):
    v = cv_ref[:]
    ix = ci_ref[:]
    vi = v[:, :, None]
    vj = v[:, None, :]
    ii = ix[:, :, None]
    ij = ix[:, None, :]
    beats = jnp.logical_or(vj > vi, jnp.logical_and(vj == vi, ij < ii))
    rank = jnp.sum(beats.astype(jnp.int32), axis=2)
    k = lax.broadcasted_iota(jnp.int32, (RB, 128, TOPK), 2)
    oh = (rank[:, :, None] == k)
    sv = jnp.sum(jnp.where(oh, v[:, :, None], 0.0), axis=1)
    si = jnp.sum(jnp.where(oh, ix[:, :, None], 0), axis=1)
    m = jnp.max(sv, axis=1, keepdims=True)
    e = jnp.exp(sv - m)
    s = jnp.sum(e, axis=1, keepdims=True)
    w_ref[:] = e / s
    idx_ref[:] = si


def _rank_softmax(cand_v, cand_i):
    return pl.pallas_call(
        _rank_kernel,
        grid=(BATCH // RB,),
        in_specs=[
            pl.BlockSpec((RB, OUT_CAP), lambda i: (i, 0)),
            pl.BlockSpec((RB, OUT_CAP), lambda i: (i, 0)),
        ],
        out_specs=[
            pl.BlockSpec((RB, TOPK), lambda i: (i, 0)),
            pl.BlockSpec((RB, TOPK), lambda i: (i, 0)),
        ],
        out_shape=[
            jax.ShapeDtypeStruct((BATCH, TOPK), jnp.float32),
            jax.ShapeDtypeStruct((BATCH, TOPK), jnp.int32),
        ],
    )(cand_v, cand_i)


def kernel(x, t, W0, b0, W1, b1, W2, b2, W3, b3, keys):
    q = _compute_q(x, t, W0, b0, W1, b1, W2, b2, W3, b3)
    keys_pad = jnp.pad(keys, ((0, N_PAD - N_TRAIN), (0, 0)))
    logits3 = _compute_logits3(q, keys_pad)
    cand_v, cand_i = _sc_topk(logits3)
    weights, top_idx = _rank_softmax(cand_v, cand_i)
    return weights, top_idx
